# Initial kernel scaffold; baseline (speedup 1.0000x reference)
import jax
import jax.numpy as jnp
from jax.experimental import pallas as pl


def kernel(z, codebook):
    raise NotImplementedError("write your pallas kernel here")


# speedup vs baseline: 1.0625x; 1.0625x over previous
import functools

import jax
import jax.numpy as jnp
from jax import lax
from jax.experimental import pallas as pl
from jax.experimental.pallas import tpu as pltpu
from jax.experimental.pallas import tpu_sc as plsc

K_TOTAL = 8192
D = 256
N_BATCH = 16
HW = 1024
K_BLK = 1024
K_STEPS = K_TOTAL // K_BLK
LOSS_SCALE = 0.25 / float(N_BATCH * HW * D)


def _dist_argmin_body(z_ref, cb_ref, sumx_ref, sumc_ref,
                      idx_ref, minval_ref, loss_ref, mv_s, mi_s):
    k = pl.program_id(1)
    b = pl.program_id(0)
    zb = z_ref[0]
    cb = cb_ref[...]
    z2 = (2.0 * zb).astype(jnp.bfloat16)
    dot2 = lax.dot_general(cb.astype(jnp.bfloat16), z2,
                           (((1,), (0,)), ((), ())),
                           preferred_element_type=jnp.float32)
    sumx = sumx_ref[0]
    sumc = sumc_ref[...]
    d2 = (sumx - dot2) + sumc
    dist = jnp.sqrt(jnp.maximum(d2, 0.0))
    bmin = jnp.min(dist, axis=0, keepdims=True)
    iota = lax.broadcasted_iota(jnp.int32, dist.shape, 0)
    bidx = jnp.min(jnp.where(dist == bmin, iota, K_BLK), axis=0) + k * K_BLK

    @pl.when(k == 0)
    def _():
        mv_s[0, :] = bmin[0].astype(jnp.bfloat16)
        mi_s[0, :] = bidx

    @pl.when(k > 0)
    def _():
        prev = mv_s[0, :].astype(jnp.float32)
        better = bmin[0] < prev
        mv_s[0, :] = jnp.where(better, bmin[0], prev).astype(jnp.bfloat16)
        mi_s[0, :] = jnp.where(better, bidx, mi_s[0, :])

    @pl.when((b == 0) & (k == 0))
    def _():
        loss_ref[0, 0] = 0.0

    @pl.when(k == K_STEPS - 1)
    def _():
        mv = mv_s[0, :].astype(jnp.float32)
        minval_ref[0, 0, :] = mv
        idx_ref[0, 0, :] = mi_s[0, :]
        part = jnp.sum(mv * mv)
        @pl.when(b == N_BATCH - 1)
        def _():
            loss_ref[0, 0] = (loss_ref[0, 0] + part) * LOSS_SCALE
        @pl.when(b < N_BATCH - 1)
        def _():
            loss_ref[0, 0] = loss_ref[0, 0] + part


def _dist_argmin(zr, codebook, sumx, sumc):
    return pl.pallas_call(
        _dist_argmin_body,
        grid=(N_BATCH, K_STEPS),
        in_specs=[
            pl.BlockSpec((1, D, HW), lambda b, k: (b, 0, 0)),
            pl.BlockSpec((K_BLK, D), lambda b, k: (k, 0)),
            pl.BlockSpec((1, 1, HW), lambda b, k: (b, 0, 0)),
            pl.BlockSpec((K_BLK, 1), lambda b, k: (k, 0)),
        ],
        out_specs=[
            pl.BlockSpec((1, 1, HW), lambda b, k: (b, 0, 0)),
            pl.BlockSpec((1, 1, HW), lambda b, k: (b, 0, 0)),
            pl.BlockSpec((1, 1), lambda b, k: (0, 0),
                         memory_space=pltpu.SMEM),
        ],
        out_shape=[
            jax.ShapeDtypeStruct((N_BATCH, 1, HW), jnp.int32),
            jax.ShapeDtypeStruct((N_BATCH, 1, HW), jnp.float32),
            jax.ShapeDtypeStruct((1, 1), jnp.float32),
        ],
        scratch_shapes=[
            pltpu.VMEM((8, HW), jnp.bfloat16),
            pltpu.VMEM((8, HW), jnp.int32),
        ],
        compiler_params=pltpu.CompilerParams(
            dimension_semantics=("arbitrary", "arbitrary"),
        ),
    )(zr, codebook, sumx, sumc)


_NW = 32
_ROWS_PER_W = (N_BATCH * HW) // _NW
_CHUNK = 128
_NCHUNK = _ROWS_PER_W // _CHUNK


def _sc_gather(codebook, idx3):
    mesh = plsc.VectorSubcoreMesh(core_axis_name="c", subcore_axis_name="s")

    @functools.partial(
        pl.kernel,
        mesh=mesh,
        out_type=jax.ShapeDtypeStruct((N_BATCH * HW, D), jnp.float32),
        scratch_types=[
            pltpu.VMEM((_NCHUNK, _CHUNK), jnp.int32),
            pltpu.VMEM((_CHUNK, D), jnp.float32),
            pltpu.SemaphoreType.DMA,
        ],
    )
    def gather_k(cb_hbm, idx_hbm, out_hbm, idx_v, rows_v, sem):
        wid = lax.axis_index("s") * 2 + lax.axis_index("c")
        base = wid * _ROWS_PER_W
        pltpu.sync_copy(idx_hbm.at[wid], idx_v)
        for j in range(_NCHUNK):
            pltpu.async_copy(cb_hbm.at[idx_v.at[j]], rows_v, sem).wait()
            pltpu.sync_copy(rows_v, out_hbm.at[pl.ds(base + j * _CHUNK, _CHUNK)])

    return gather_k(codebook, idx3)


def kernel(z, codebook):
    zr = z.reshape(N_BATCH, D, HW)
    sumx = jnp.sum(z * z, axis=1).reshape(N_BATCH, 1, HW)
    sumc = jnp.sum(codebook * codebook, axis=1)[:, None]
    idx, _minval, loss = _dist_argmin(zr, codebook, sumx, sumc)
    idx_w = idx.reshape(_NW, _NCHUNK, _CHUNK)
    rows = _sc_gather(codebook, idx_w)
    zq_out = rows.reshape(N_BATCH, 32, 32, D).transpose(0, 3, 1, 2)
    code_indices = idx.reshape(N_BATCH, 32, 32)
    return (loss[0, 0], zq_out, code_indices)
